# baseline (speedup 1.0000x reference)
import functools

import jax
import jax.numpy as jnp
from jax import lax
from jax.experimental import pallas as pl
from jax.experimental.pallas import tpu as pltpu
from jax.experimental.pallas import tpu_sc as plsc

_TOK_BLOCK = 512


def _dist_argmin_kernel(x_ref, ctm_ref, idx_ref, loss_ref):
    x = x_ref[...]
    ctm = ctm_ref[...]
    k = ctm.shape[1]
    half = k // 2
    x2 = jnp.sum(x * x, axis=-1, keepdims=True)
    c2 = 0.25 * jnp.sum(ctm * ctm, axis=0, keepdims=True)
    xcm = jnp.dot(
        x.astype(jnp.bfloat16),
        ctm.astype(jnp.bfloat16),
        preferred_element_type=jnp.float32,
    )

    t = x.shape[0]
    lanes = 128
    nblk = k // lanes

    def scan_half(lo, hi):
        run = jnp.full((t, lanes), jnp.inf, dtype=jnp.float32)
        jwin = jnp.zeros((t, lanes), dtype=jnp.int32)
        for b in range(lo, hi):
            sl = slice(b * lanes, (b + 1) * lanes)
            d2b = (x2 + c2[:, sl]) + xcm[:, sl]
            m = d2b < run
            run = jnp.minimum(run, d2b)
            jwin = jnp.where(m, b, jwin)
        ids = jwin * lanes + lax.broadcasted_iota(jnp.int32, (t, lanes), 1)
        mv = jnp.min(run, axis=-1, keepdims=True)
        iv = jnp.min(jnp.where(run <= mv, ids, k), axis=-1)
        return mv, iv

    ma, ia = scan_half(0, nblk // 2)
    mb, ib = scan_half(nblk // 2, nblk)
    da = jnp.sqrt(jnp.maximum(ma, 0.0))
    db = jnp.sqrt(jnp.maximum(mb, 0.0))
    dab = da.astype(jnp.bfloat16).astype(jnp.float32)
    pick_a = (dab <= db)[:, 0]
    idx_ref[...] = jnp.where(pick_a, ia, ib)

    @pl.when(pl.program_id(0) == 0)
    def _init():
        loss_ref[0, 0] = 0.0

    loss_ref[0, 0] += jnp.sum(jnp.maximum(jnp.minimum(ma, mb), 0.0))


def _dist_argmin(flat, codebook):
    n, d = flat.shape
    kk = codebook.shape[0]
    nb = n // _TOK_BLOCK
    idx_flat, loss_sum = pl.pallas_call(
        _dist_argmin_kernel,
        grid=(nb,),
        in_specs=[
            pl.BlockSpec((_TOK_BLOCK, d), lambda i: (i, 0)),
            pl.BlockSpec((d, kk), lambda i: (0, 0)),
        ],
        out_specs=[
            pl.BlockSpec((_TOK_BLOCK,), lambda i: (i,)),
            pl.BlockSpec(memory_space=pltpu.SMEM),
        ],
        out_shape=[
            jax.ShapeDtypeStruct((n,), jnp.int32),
            jax.ShapeDtypeStruct((1, 1), jnp.float32),
        ],
    )(flat, codebook.T * -2.0)
    return idx_flat, loss_sum


def _sc_gather(codebook, idx_flat, n, d):
    info = plsc.get_sparse_core_info()
    nw = info.num_cores * info.num_subcores
    ch = n // nw
    nch = 1
    idx3 = idx_flat.reshape(nw, nch, ch)
    w = 128
    table = jnp.pad(codebook, ((0, 0), (0, w - d)))
    mesh = plsc.VectorSubcoreMesh(core_axis_name="c", subcore_axis_name="s")

    @functools.partial(
        pl.kernel,
        mesh=mesh,
        out_type=jax.ShapeDtypeStruct((nw, nch, ch, w), jnp.float32),
        scratch_types=[
            pltpu.VMEM((nch, ch), jnp.int32),
            pltpu.VMEM((nch, ch, w), jnp.float32),
            pltpu.SemaphoreType.DMA,
        ],
    )
    def gather_k(table_hbm, idx_hbm, out_hbm, idx_v, rows_v, sem):
        wid = lax.axis_index("s") * info.num_cores + lax.axis_index("c")
        pltpu.sync_copy(idx_hbm.at[wid], idx_v)
        copies = [
            pltpu.async_copy(table_hbm.at[idx_v.at[j]], rows_v.at[j], sem)
            for j in range(nch)
        ]
        for c in copies:
            c.wait()
        pltpu.sync_copy(rows_v, out_hbm.at[wid])

    return gather_k(table, idx3).reshape(n, w)[:, :d]


def kernel(x, codebook):
    b, t, d = x.shape
    n = b * t
    flat = x.reshape(n, d)
    h = n // 2
    idx0, loss0 = _dist_argmin(flat[:h], codebook)
    q0 = _sc_gather(codebook, idx0, h, d)
    idx1, loss1 = _dist_argmin(flat[h:], codebook)
    q1 = _sc_gather(codebook, idx1, h, d)
    quantized = jnp.concatenate([q0, q1], axis=0).reshape(x.shape)
    indices = jnp.concatenate([idx0, idx1], axis=0).reshape(b, t)
    commit_loss = (loss0[0, 0] + loss1[0, 0]) / (n * d)
    return (quantized, indices, commit_loss)


# speedup vs baseline: 1.0689x; 1.0689x over previous
import functools

import jax
import jax.numpy as jnp
from jax import lax
from jax.experimental import pallas as pl
from jax.experimental.pallas import tpu as pltpu
from jax.experimental.pallas import tpu_sc as plsc

_TOK_BLOCK = 1024


def _dist_argmin_kernel(x_ref, ctm_ref, idx_ref, loss_ref):
    x = x_ref[...]
    ctm = ctm_ref[...]
    k = ctm.shape[1]
    half = k // 2
    x2 = jnp.sum(x * x, axis=-1, keepdims=True)
    c2 = 0.25 * jnp.sum(ctm * ctm, axis=0, keepdims=True)
    xcm = jnp.dot(
        x.astype(jnp.bfloat16),
        ctm.astype(jnp.bfloat16),
        preferred_element_type=jnp.float32,
    )

    t = x.shape[0]
    lanes = 128
    nblk = k // lanes

    def scan_half(lo, hi):
        run = jnp.full((t, lanes), jnp.inf, dtype=jnp.float32)
        jwin = jnp.zeros((t, lanes), dtype=jnp.int32)
        for b in range(lo, hi):
            sl = slice(b * lanes, (b + 1) * lanes)
            d2b = (x2 + c2[:, sl]) + xcm[:, sl]
            m = d2b < run
            run = jnp.minimum(run, d2b)
            jwin = jnp.where(m, b, jwin)
        ids = jwin * lanes + lax.broadcasted_iota(jnp.int32, (t, lanes), 1)
        mv = jnp.min(run, axis=-1, keepdims=True)
        iv = jnp.min(jnp.where(run <= mv, ids, k), axis=-1)
        return mv, iv

    ma, ia = scan_half(0, nblk // 2)
    mb, ib = scan_half(nblk // 2, nblk)
    da = jnp.sqrt(jnp.maximum(ma, 0.0))
    db = jnp.sqrt(jnp.maximum(mb, 0.0))
    dab = da.astype(jnp.bfloat16).astype(jnp.float32)
    pick_a = (dab <= db)[:, 0]
    idx_ref[...] = jnp.where(pick_a, ia, ib)

    @pl.when(pl.program_id(0) == 0)
    def _init():
        loss_ref[0, 0] = 0.0

    loss_ref[0, 0] += jnp.sum(jnp.maximum(jnp.minimum(ma, mb), 0.0))


def _dist_argmin(flat, codebook):
    n, d = flat.shape
    kk = codebook.shape[0]
    nb = n // _TOK_BLOCK
    idx_flat, loss_sum = pl.pallas_call(
        _dist_argmin_kernel,
        grid=(nb,),
        in_specs=[
            pl.BlockSpec((_TOK_BLOCK, d), lambda i: (i, 0)),
            pl.BlockSpec((d, kk), lambda i: (0, 0)),
        ],
        out_specs=[
            pl.BlockSpec((_TOK_BLOCK,), lambda i: (i,)),
            pl.BlockSpec(memory_space=pltpu.SMEM),
        ],
        out_shape=[
            jax.ShapeDtypeStruct((n,), jnp.int32),
            jax.ShapeDtypeStruct((1, 1), jnp.float32),
        ],
    )(flat, codebook.T * -2.0)
    return idx_flat, loss_sum


def _sc_gather(codebook, idx_flat, n, d):
    info = plsc.get_sparse_core_info()
    nw = info.num_cores * info.num_subcores
    ch = n // nw
    nch = 1
    idx3 = idx_flat.reshape(nw, nch, ch)
    w = 128
    table = jnp.pad(codebook, ((0, 0), (0, w - d)))
    mesh = plsc.VectorSubcoreMesh(core_axis_name="c", subcore_axis_name="s")

    @functools.partial(
        pl.kernel,
        mesh=mesh,
        out_type=jax.ShapeDtypeStruct((nw, nch, ch, w), jnp.float32),
        scratch_types=[
            pltpu.VMEM((nch, ch), jnp.int32),
            pltpu.VMEM((nch, ch, w), jnp.float32),
            pltpu.SemaphoreType.DMA,
        ],
    )
    def gather_k(table_hbm, idx_hbm, out_hbm, idx_v, rows_v, sem):
        wid = lax.axis_index("s") * info.num_cores + lax.axis_index("c")
        pltpu.sync_copy(idx_hbm.at[wid], idx_v)
        copies = [
            pltpu.async_copy(table_hbm.at[idx_v.at[j]], rows_v.at[j], sem)
            for j in range(nch)
        ]
        for c in copies:
            c.wait()
        pltpu.sync_copy(rows_v, out_hbm.at[wid])

    return gather_k(table, idx3).reshape(n, w)[:, :d]


def kernel(x, codebook):
    b, t, d = x.shape
    n = b * t
    flat = x.reshape(n, d)
    idx_flat, loss_sum = _dist_argmin(flat, codebook)
    quantized = _sc_gather(codebook, idx_flat, n, d).reshape(x.shape)
    indices = idx_flat.reshape(b, t)
    commit_loss = loss_sum[0, 0] / (n * d)
    return (quantized, indices, commit_loss)
